# baseline (speedup 1.0000x reference)
import functools

import jax
import jax.numpy as jnp
from jax import lax
from jax.experimental import pallas as pl
from jax.experimental.pallas import tpu as pltpu
from jax.experimental.pallas import tpu_sc as plsc

VOCAB_SIZE = 1000000
EMBED_DIM = 64
CTX_LEN = 200

NUM_WORKERS = 32
ROWS_PER_WORKER = 8
PADDED_CTX = NUM_WORKERS * ROWS_PER_WORKER
TILE_ROWS = 8
NUM_TILES_TABLE = VOCAB_SIZE // TILE_ROWS

V_TILE = 8000
N_TILES = VOCAB_SIZE // V_TILE


def _sc_gather_partials(table_pairs, pair_ids, halfb):
    mesh = plsc.VectorSubcoreMesh(core_axis_name="c", subcore_axis_name="s")

    @functools.partial(
        pl.kernel,
        mesh=mesh,
        out_type=jax.ShapeDtypeStruct((NUM_WORKERS, EMBED_DIM), jnp.float32),
        scratch_types=[
            pltpu.VMEM((ROWS_PER_WORKER,), jnp.int32),
            pltpu.VMEM((ROWS_PER_WORKER, 16), jnp.int32),
            pltpu.VMEM((ROWS_PER_WORKER, 2 * EMBED_DIM), jnp.float32),
            pltpu.VMEM((EMBED_DIM,), jnp.float32),
            pltpu.SemaphoreType.DMA,
        ],
    )
    def gather_kernel(
        table_hbm, pid_hbm, halfb_hbm, out_hbm, pid_v, halfb_v, rows_v, acc_v, sem
    ):
        num_cores = 2
        wid = lax.axis_index("s") * num_cores + lax.axis_index("c")
        base = wid * ROWS_PER_WORKER
        pltpu.sync_copy(pid_hbm.at[pl.ds(base, ROWS_PER_WORKER)], pid_v)
        pltpu.sync_copy(halfb_hbm.at[pl.ds(base, ROWS_PER_WORKER)], halfb_v)
        pltpu.async_copy(table_hbm.at[pid_v], rows_v, sem).wait()

        num_groups = EMBED_DIM // 16
        zero16 = jnp.zeros((16,), jnp.float32)
        accs = [zero16 for _ in range(num_groups)]
        for j in range(ROWS_PER_WORKER):
            w = jnp.where(base + j < CTX_LEN, 1.0, 0.0).astype(jnp.float32)
            hv = halfb_v[j, :]
            for g in range(num_groups):
                v0 = rows_v[j, pl.ds(g * 16, 16)]
                v1 = rows_v[j, pl.ds(EMBED_DIM + g * 16, 16)]
                accs[g] = accs[g] + jnp.where(hv == 0, v0, v1) * w
        for g in range(num_groups):
            acc_v[pl.ds(g * 16, 16)] = accs[g]
        pltpu.sync_copy(acc_v, out_hbm.at[wid])

    return gather_kernel(table_pairs, pair_ids, halfb)


def _tc_body(part_ref, w_ref, b_ref, out_ref, q_s, m_s, l_s):
    i = pl.program_id(0)

    @pl.when(i == 0)
    def _init():
        q_s[:, :] = jnp.sum(part_ref[:, :], axis=0, keepdims=True) * (
            1.0 / CTX_LEN
        )
        m_s[0] = -jnp.inf
        l_s[0] = 0.0

    q = q_s[:, :]
    w = w_ref[:, :]
    r = lax.dot_general(
        q, w, (((1,), (1,)), ((), ())), preferred_element_type=jnp.float32
    )
    r = r + b_ref[pl.ds(i, 1), :]
    out_ref[pl.ds(i, 1), :] = r

    m_old = m_s[0]
    m_new = jnp.maximum(m_old, jnp.max(r))
    l_s[0] = l_s[0] * jnp.exp(m_old - m_new) + jnp.sum(jnp.exp(r - m_new))
    m_s[0] = m_new

    @pl.when(i == N_TILES - 1)
    def _finish():
        lse = m_s[0] + jnp.log(l_s[0])
        out_ref[:, :] = out_ref[:, :] - lse


def _tc_logits(partials, W, b2):
    return pl.pallas_call(
        _tc_body,
        grid=(N_TILES,),
        in_specs=[
            pl.BlockSpec((NUM_WORKERS, EMBED_DIM), lambda i: (0, 0)),
            pl.BlockSpec((V_TILE, EMBED_DIM), lambda i: (i, 0)),
            pl.BlockSpec((N_TILES, V_TILE), lambda i: (0, 0)),
        ],
        out_specs=pl.BlockSpec((N_TILES, V_TILE), lambda i: (0, 0)),
        out_shape=jax.ShapeDtypeStruct((N_TILES, V_TILE), jnp.float32),
        scratch_shapes=[
            pltpu.VMEM((1, EMBED_DIM), jnp.float32),
            pltpu.SMEM((1,), jnp.float32),
            pltpu.SMEM((1,), jnp.float32),
        ],
    )(partials, W, b2)


def kernel(X, emb_table, W, b):
    idx_padded = jnp.concatenate(
        [X.astype(jnp.int32), jnp.zeros((PADDED_CTX - CTX_LEN,), jnp.int32)]
    )
    table_pairs = emb_table.reshape(VOCAB_SIZE // 2, 2 * EMBED_DIM)
    pair_ids = idx_padded // 2
    halfb = jnp.broadcast_to(
        (idx_padded % 2)[:, None], (PADDED_CTX, 16)
    ).astype(jnp.int32)
    partials = _sc_gather_partials(table_pairs, pair_ids, halfb)
    b2 = b.reshape(N_TILES, V_TILE)
    s2 = _tc_logits(partials, W, b2)
    return s2.reshape(1, VOCAB_SIZE)


# speedup vs baseline: 1.3100x; 1.3100x over previous
import jax
import jax.numpy as jnp
from jax import lax
from jax.experimental import pallas as pl
from jax.experimental.pallas import tpu as pltpu

VOCAB_SIZE = 1000000
EMBED_DIM = 64
CTX_LEN = 200

V_TILE = 8000
N_TILES = VOCAB_SIZE // V_TILE


def _body(x_ref, emb_ref, w_ref, b_ref, out_ref, rows_v, q_s, m_s, l_s, gsem):
    i = pl.program_id(0)

    @pl.when(i == 0)
    def _init():
        def issue(j, carry):
            idx = x_ref[j]
            pltpu.make_async_copy(
                emb_ref.at[pl.ds(idx, 1), :], rows_v.at[pl.ds(j, 1), :], gsem
            ).start()
            return carry

        lax.fori_loop(0, CTX_LEN, issue, 0)

        def drain(j, carry):
            pltpu.make_async_copy(
                emb_ref.at[pl.ds(0, 1), :], rows_v.at[pl.ds(0, 1), :], gsem
            ).wait()
            return carry

        lax.fori_loop(0, CTX_LEN, drain, 0)

        q_s[:, :] = jnp.sum(rows_v[:, :], axis=0, keepdims=True) * (
            1.0 / CTX_LEN
        )
        m_s[0] = -jnp.inf
        l_s[0] = 0.0

    q = q_s[:, :]
    w = w_ref[:, :]
    r = lax.dot_general(
        q, w, (((1,), (1,)), ((), ())), preferred_element_type=jnp.float32
    )
    r = r + b_ref[pl.ds(i, 1), :]
    out_ref[pl.ds(i, 1), :] = r

    m_old = m_s[0]
    m_new = jnp.maximum(m_old, jnp.max(r))
    l_s[0] = l_s[0] * jnp.exp(m_old - m_new) + jnp.sum(jnp.exp(r - m_new))
    m_s[0] = m_new

    @pl.when(i == N_TILES - 1)
    def _finish():
        lse = m_s[0] + jnp.log(l_s[0])
        out_ref[:, :] = out_ref[:, :] - lse


def kernel(X, emb_table, W, b):
    b2 = b.reshape(N_TILES, V_TILE)
    s2 = pl.pallas_call(
        _body,
        grid=(N_TILES,),
        in_specs=[
            pl.BlockSpec(memory_space=pltpu.SMEM),
            pl.BlockSpec(memory_space=pl.ANY),
            pl.BlockSpec((V_TILE, EMBED_DIM), lambda i: (i, 0)),
            pl.BlockSpec((N_TILES, V_TILE), lambda i: (0, 0)),
        ],
        out_specs=pl.BlockSpec((N_TILES, V_TILE), lambda i: (0, 0)),
        out_shape=jax.ShapeDtypeStruct((N_TILES, V_TILE), jnp.float32),
        scratch_shapes=[
            pltpu.VMEM((CTX_LEN, EMBED_DIM), jnp.float32),
            pltpu.VMEM((1, EMBED_DIM), jnp.float32),
            pltpu.SMEM((1,), jnp.float32),
            pltpu.SMEM((1,), jnp.float32),
            pltpu.SemaphoreType.DMA,
        ],
    )(X.astype(jnp.int32), emb_table, W, b2)
    return s2.reshape(1, VOCAB_SIZE)


# speedup vs baseline: 1.3890x; 1.0603x over previous
import jax
import jax.numpy as jnp
from jax import lax
from jax.experimental import pallas as pl
from jax.experimental.pallas import tpu as pltpu

VOCAB_SIZE = 1000000
EMBED_DIM = 64
CTX_LEN = 200

V_TILE = 40000
N_TILES = VOCAB_SIZE // V_TILE


def _body(x_ref, emb_ref, w_ref, b_ref, out_ref, rows_v, q_s, m_s, l_s, gsem):
    i = pl.program_id(0)

    @pl.when(i == 0)
    def _init():
        def issue(j, carry):
            idx = x_ref[j]
            pltpu.make_async_copy(
                emb_ref.at[pl.ds(idx, 1), :], rows_v.at[pl.ds(j, 1), :], gsem
            ).start()
            return carry

        lax.fori_loop(0, CTX_LEN, issue, 0)

        def drain(j, carry):
            pltpu.make_async_copy(
                emb_ref.at[pl.ds(0, 1), :], rows_v.at[pl.ds(0, 1), :], gsem
            ).wait()
            return carry

        lax.fori_loop(0, CTX_LEN, drain, 0)

        q_s[:, :] = jnp.sum(rows_v[:, :], axis=0, keepdims=True) * (
            1.0 / CTX_LEN
        )
        m_s[0] = -jnp.inf
        l_s[0] = 0.0

    q = q_s[:, :]
    w = w_ref[:, :]
    r = lax.dot_general(
        q, w, (((1,), (1,)), ((), ())), preferred_element_type=jnp.float32
    )
    r = r + b_ref[pl.ds(i, 1), :]
    out_ref[pl.ds(i, 1), :] = r

    m_old = m_s[0]
    m_new = jnp.maximum(m_old, jnp.max(r))
    l_s[0] = l_s[0] * jnp.exp(m_old - m_new) + jnp.sum(jnp.exp(r - m_new))
    m_s[0] = m_new

    @pl.when(i == N_TILES - 1)
    def _finish():
        lse = m_s[0] + jnp.log(l_s[0])
        out_ref[:, :] = out_ref[:, :] - lse


def kernel(X, emb_table, W, b):
    b2 = b.reshape(N_TILES, V_TILE)
    s2 = pl.pallas_call(
        _body,
        grid=(N_TILES,),
        in_specs=[
            pl.BlockSpec(memory_space=pltpu.SMEM),
            pl.BlockSpec(memory_space=pl.ANY),
            pl.BlockSpec((V_TILE, EMBED_DIM), lambda i: (i, 0)),
            pl.BlockSpec((N_TILES, V_TILE), lambda i: (0, 0)),
        ],
        out_specs=pl.BlockSpec((N_TILES, V_TILE), lambda i: (0, 0)),
        out_shape=jax.ShapeDtypeStruct((N_TILES, V_TILE), jnp.float32),
        scratch_shapes=[
            pltpu.VMEM((CTX_LEN, EMBED_DIM), jnp.float32),
            pltpu.VMEM((1, EMBED_DIM), jnp.float32),
            pltpu.SMEM((1,), jnp.float32),
            pltpu.SMEM((1,), jnp.float32),
            pltpu.SemaphoreType.DMA,
        ],
    )(X.astype(jnp.int32), emb_table, W, b2)
    return s2.reshape(1, VOCAB_SIZE)


# speedup vs baseline: 1.3928x; 1.0027x over previous
import jax
import jax.numpy as jnp
from jax import lax
from jax.experimental import pallas as pl
from jax.experimental.pallas import tpu as pltpu

VOCAB_SIZE = 1000000
EMBED_DIM = 64
CTX_LEN = 200

V_TILE = 8000
N_TILES = VOCAB_SIZE // V_TILE
NBUF = 10


def _body(x_ref, emb_ref, w_ref, b_ref, out_ref, rows_v, w_buf, gsem, wsems):
    def issue_g(j, carry):
        idx = x_ref[j]
        pltpu.make_async_copy(
            emb_ref.at[pl.ds(idx, 1), :], rows_v.at[pl.ds(j, 1), :], gsem
        ).start()
        return carry

    lax.fori_loop(0, CTX_LEN, issue_g, 0)

    def w_copy(t):
        slot = lax.rem(t, NBUF)
        return pltpu.make_async_copy(
            w_ref.at[pl.ds(t * V_TILE, V_TILE), :],
            w_buf.at[pl.ds(slot * V_TILE, V_TILE), :],
            wsems.at[slot],
        )

    def issue_w(t, carry):
        w_copy(t).start()
        return carry

    lax.fori_loop(0, NBUF - 1, issue_w, 0)

    def drain_g(j, carry):
        pltpu.make_async_copy(
            emb_ref.at[pl.ds(0, 1), :], rows_v.at[pl.ds(0, 1), :], gsem
        ).wait()
        return carry

    lax.fori_loop(0, CTX_LEN, drain_g, 0)
    q = jnp.sum(rows_v[:, :], axis=0, keepdims=True) * (1.0 / CTX_LEN)

    def step(t, carry):
        m, l = carry
        slot = lax.rem(t, NBUF)
        w_copy(t).wait()
        w = w_buf[pl.ds(slot * V_TILE, V_TILE), :]
        r = lax.dot_general(
            q, w, (((1,), (1,)), ((), ())), preferred_element_type=jnp.float32
        )
        r = r + b_ref[pl.ds(t, 1), :]
        out_ref[pl.ds(t, 1), :] = r
        m_new = jnp.maximum(m, jnp.max(r))
        l = l * jnp.exp(m - m_new) + jnp.sum(jnp.exp(r - m_new))

        nxt = t + NBUF - 1

        @pl.when(nxt < N_TILES)
        def _():
            w_copy(nxt).start()

        return (m_new, l)

    m, l = lax.fori_loop(
        0, N_TILES, step, (jnp.float32(-jnp.inf), jnp.float32(0.0))
    )
    lse = m + jnp.log(l)
    out_ref[:, :] = out_ref[:, :] - lse


def kernel(X, emb_table, W, b):
    b2 = b.reshape(N_TILES, V_TILE)
    s2 = pl.pallas_call(
        _body,
        in_specs=[
            pl.BlockSpec(memory_space=pltpu.SMEM),
            pl.BlockSpec(memory_space=pl.ANY),
            pl.BlockSpec(memory_space=pl.ANY),
            pl.BlockSpec(memory_space=pltpu.VMEM),
        ],
        out_specs=pl.BlockSpec(memory_space=pltpu.VMEM),
        out_shape=jax.ShapeDtypeStruct((N_TILES, V_TILE), jnp.float32),
        scratch_shapes=[
            pltpu.VMEM((CTX_LEN, EMBED_DIM), jnp.float32),
            pltpu.VMEM((NBUF * V_TILE, EMBED_DIM), jnp.float32),
            pltpu.SemaphoreType.DMA,
            pltpu.SemaphoreType.DMA((NBUF,)),
        ],
    )(X.astype(jnp.int32), emb_table, W, b2)
    return s2.reshape(1, VOCAB_SIZE)
